# baseline (speedup 1.0000x reference)
import jax
import jax.numpy as jnp
from jax.experimental import pallas as pl
from jax.experimental.pallas import tpu as pltpu

_BM = 1024
_CP = 128


def _mlp_kernel(x_ref, w1_ref, b1_ref, w2_ref, b2_ref, o_ref):
    a = jnp.dot(x_ref[...], w1_ref[...], preferred_element_type=jnp.float32)
    h = jax.nn.gelu(a + b1_ref[...])
    o = jnp.dot(h.astype(jnp.bfloat16), w2_ref[...],
                preferred_element_type=jnp.float32)
    o_ref[...] = o + b2_ref[...]


def kernel(data, inits, W1, b1, W2, b2):
    b, t, d = data.shape
    h_dim = W1.shape[1]
    c = W2.shape[1]
    n = b * t

    x = data.reshape(n, d).astype(jnp.bfloat16)
    w1 = W1.astype(jnp.bfloat16)
    w2 = jnp.zeros((h_dim, _CP), jnp.bfloat16).at[:, :c].set(
        W2.astype(jnp.bfloat16))
    b1r = b1.reshape(1, h_dim)
    b2r = jnp.zeros((1, _CP), jnp.float32).at[0, :c].set(b2)

    out = pl.pallas_call(
        _mlp_kernel,
        grid=(n // _BM,),
        in_specs=[
            pl.BlockSpec((_BM, d), lambda i: (i, 0)),
            pl.BlockSpec((d, h_dim), lambda i: (0, 0)),
            pl.BlockSpec((1, h_dim), lambda i: (0, 0)),
            pl.BlockSpec((h_dim, _CP), lambda i: (0, 0)),
            pl.BlockSpec((1, _CP), lambda i: (0, 0)),
        ],
        out_specs=pl.BlockSpec((_BM, _CP), lambda i: (i, 0)),
        out_shape=jax.ShapeDtypeStruct((n, _CP), jnp.float32),
        compiler_params=pltpu.CompilerParams(
            dimension_semantics=("parallel",)),
    )(x, w1, b1r, w2, b2r)
    return out[:, :c]


# speedup vs baseline: 1.1494x; 1.1494x over previous
import jax
import jax.numpy as jnp
from jax.experimental import pallas as pl
from jax.experimental.pallas import tpu as pltpu

_BM = 1024
_CP = 128


def _mlp_kernel(x_ref, w1_ref, b1_ref, w2_ref, b2_ref, o_ref):
    a = jnp.dot(x_ref[...].astype(jnp.bfloat16), w1_ref[...],
                preferred_element_type=jnp.float32)
    h = jax.nn.gelu(a + b1_ref[...])
    o = jnp.dot(h.astype(jnp.bfloat16), w2_ref[...],
                preferred_element_type=jnp.float32)
    o_ref[...] = o + b2_ref[...]


def kernel(data, inits, W1, b1, W2, b2):
    b, t, d = data.shape
    h_dim = W1.shape[1]
    c = W2.shape[1]
    n = b * t

    x = data.reshape(n, d)
    w1 = W1.astype(jnp.bfloat16)
    w2 = jnp.zeros((h_dim, _CP), jnp.bfloat16).at[:, :c].set(
        W2.astype(jnp.bfloat16))
    b1r = b1.reshape(1, h_dim)
    b2r = jnp.zeros((1, _CP), jnp.float32).at[0, :c].set(b2)

    out = pl.pallas_call(
        _mlp_kernel,
        grid=(n // _BM,),
        in_specs=[
            pl.BlockSpec((_BM, d), lambda i: (i, 0)),
            pl.BlockSpec((d, h_dim), lambda i: (0, 0)),
            pl.BlockSpec((1, h_dim), lambda i: (0, 0)),
            pl.BlockSpec((h_dim, _CP), lambda i: (0, 0)),
            pl.BlockSpec((1, _CP), lambda i: (0, 0)),
        ],
        out_specs=pl.BlockSpec((_BM, _CP), lambda i: (i, 0)),
        out_shape=jax.ShapeDtypeStruct((n, _CP), jnp.float32),
        compiler_params=pltpu.CompilerParams(
            dimension_semantics=("parallel",)),
    )(x, w1, b1r, w2, b2r)
    return out[:, :c]
